# baseline (speedup 1.0000x reference)
import jax
import jax.numpy as jnp
from jax import lax
from jax.experimental import pallas as pl
from jax.experimental.pallas import tpu as pltpu
from jax.experimental.pallas import tpu_sc as plsc

_ROW = 200 * 64
_B = 4096
_N_TC = 2816
_TC_BLOCK = 128

_NC = 2
_NS = 16
_NW = _NC * _NS
_RPW = (_B - _N_TC) // _NW
_CR = 8
_CC = _ROW // 2
_COLS_SPLIT = _ROW // _CC
_CHUNKS = (_RPW // _CR) * _COLS_SPLIT
_NBUF = 2


def _tc_add_kernel(x_ref, pe_ref, o_ref):
    o_ref[...] = x_ref[...] + pe_ref[...]


def _add_pe_chunk(buf, pe_v, col_base):
    @pl.loop(0, _CC // 16, unroll=2)
    def _(j):
        sl = pl.ds(col_base + j * 16, 16)
        pe_row = pe_v[sl]
        for r in range(_CR):
            buf[r, pl.ds(j * 16, 16)] = buf[r, pl.ds(j * 16, 16)] + pe_row


def _sc_add(x_hbm, pe_hbm, out_hbm, pe_v, bufs, gsem, ssem):
    wid = lax.axis_index("s") * _NC + lax.axis_index("c")
    row0 = _N_TC + wid * _RPW

    pltpu.sync_copy(pe_hbm, pe_v)

    def chunk_slice(ref, c, base_row):
        rb = c // _COLS_SPLIT
        h = c % _COLS_SPLIT
        return ref.at[pl.ds(base_row + rb * _CR, _CR), pl.ds(h * _CC, _CC)]

    def issue_gather(c, b):
        pltpu.async_copy(chunk_slice(x_hbm, c, row0), bufs[b], gsem[b])

    def wait_gather(b):
        pltpu.make_async_copy(
            x_hbm.at[pl.ds(0, _CR), pl.ds(0, _CC)], bufs[b], gsem[b]
        ).wait()

    def issue_scatter(c, b):
        pltpu.async_copy(bufs[b], chunk_slice(out_hbm, c, wid * _RPW), ssem[b])

    def wait_scatter(b):
        pltpu.make_async_copy(
            bufs[b], out_hbm.at[pl.ds(0, _CR), pl.ds(0, _CC)], ssem[b]
        ).wait()

    for b in range(_NBUF):
        issue_gather(b, b)

    @pl.loop(0, _CHUNKS // _NBUF - 1)
    def _(s):
        c0 = s * _NBUF
        for b in range(_NBUF):
            wait_gather(b)
            _add_pe_chunk(bufs[b], pe_v, ((c0 + b) % _COLS_SPLIT) * _CC)
            issue_scatter(c0 + b, b)
        for b in range(_NBUF):
            wait_scatter(b)
            issue_gather(c0 + _NBUF + b, b)

    c0 = _CHUNKS - _NBUF
    for b in range(_NBUF):
        wait_gather(b)
        _add_pe_chunk(bufs[b], pe_v, ((c0 + b) % _COLS_SPLIT) * _CC)
        issue_scatter(c0 + b, b)
    for b in range(_NBUF):
        wait_scatter(b)


def kernel(x, pe):
    bsz, seq_len, d_model = x.shape
    x2 = x.reshape(bsz, _ROW)
    pe1 = pe.reshape(-1)
    pe2 = pe.reshape(1, _ROW)

    sc = pl.kernel(
        _sc_add,
        out_type=jax.ShapeDtypeStruct((_B - _N_TC, _ROW), x.dtype),
        mesh=plsc.VectorSubcoreMesh(core_axis_name="c", subcore_axis_name="s"),
        compiler_params=pltpu.CompilerParams(use_tc_tiling_on_sc=True),
        scratch_types=[
            pltpu.VMEM((_ROW,), jnp.float32),
            [pltpu.VMEM((_CR, _CC), jnp.float32) for _ in range(_NBUF)],
            [pltpu.SemaphoreType.DMA for _ in range(_NBUF)],
            [pltpu.SemaphoreType.DMA for _ in range(_NBUF)],
        ],
    )
    out_sc = sc(x2, pe1)

    out_full = pl.pallas_call(
        _tc_add_kernel,
        grid=(_N_TC // _TC_BLOCK,),
        in_specs=[
            pl.BlockSpec((_TC_BLOCK, _ROW), lambda i: (i, 0)),
            pl.BlockSpec((1, _ROW), lambda i: (0, 0)),
        ],
        out_specs=pl.BlockSpec((_TC_BLOCK, _ROW), lambda i: (i, 0)),
        out_shape=jax.ShapeDtypeStruct((_B, _ROW), x.dtype),
    )(x2, pe2)

    out = jax.lax.dynamic_update_slice(out_full, out_sc, (_N_TC, 0))
    return out.reshape(bsz, seq_len, d_model)


# speedup vs baseline: 1.0321x; 1.0321x over previous
import jax
import jax.numpy as jnp
from jax import lax
from jax.experimental import pallas as pl
from jax.experimental.pallas import tpu as pltpu
from jax.experimental.pallas import tpu_sc as plsc

_ROW = 200 * 64
_B = 4096
_N_TC = 3328
_TC_BLOCK = 128

_NC = 2
_NS = 16
_NW = _NC * _NS
_RPW = (_B - _N_TC) // _NW
_CR = 8
_CC = _ROW // 2
_COLS_SPLIT = _ROW // _CC
_CHUNKS = (_RPW // _CR) * _COLS_SPLIT
_NBUF = 2


def _tc_add_kernel(x_ref, pe_ref, o_ref):
    o_ref[...] = x_ref[...] + pe_ref[...]


def _add_pe_chunk(buf, pe_v, col_base):
    @pl.loop(0, _CC // 16, unroll=2)
    def _(j):
        sl = pl.ds(col_base + j * 16, 16)
        pe_row = pe_v[sl]
        for r in range(_CR):
            buf[r, pl.ds(j * 16, 16)] = buf[r, pl.ds(j * 16, 16)] + pe_row


def _sc_add(x_hbm, pe_hbm, out_hbm, pe_v, bufs, gsem, ssem):
    wid = lax.axis_index("s") * _NC + lax.axis_index("c")
    row0 = _N_TC + wid * _RPW

    pltpu.sync_copy(pe_hbm, pe_v)

    def chunk_slice(ref, c, base_row):
        rb = c // _COLS_SPLIT
        h = c % _COLS_SPLIT
        return ref.at[pl.ds(base_row + rb * _CR, _CR), pl.ds(h * _CC, _CC)]

    def issue_gather(c, b):
        pltpu.async_copy(chunk_slice(x_hbm, c, row0), bufs[b], gsem[b])

    def wait_gather(b):
        pltpu.make_async_copy(
            x_hbm.at[pl.ds(0, _CR), pl.ds(0, _CC)], bufs[b], gsem[b]
        ).wait()

    def issue_scatter(c, b):
        pltpu.async_copy(bufs[b], chunk_slice(out_hbm, c, wid * _RPW), ssem[b])

    def wait_scatter(b):
        pltpu.make_async_copy(
            bufs[b], out_hbm.at[pl.ds(0, _CR), pl.ds(0, _CC)], ssem[b]
        ).wait()

    for b in range(_NBUF):
        issue_gather(b, b)

    @pl.loop(0, _CHUNKS // _NBUF - 1)
    def _(s):
        c0 = s * _NBUF
        for b in range(_NBUF):
            wait_gather(b)
            _add_pe_chunk(bufs[b], pe_v, ((c0 + b) % _COLS_SPLIT) * _CC)
            issue_scatter(c0 + b, b)
        for b in range(_NBUF):
            wait_scatter(b)
            issue_gather(c0 + _NBUF + b, b)

    c0 = _CHUNKS - _NBUF
    for b in range(_NBUF):
        wait_gather(b)
        _add_pe_chunk(bufs[b], pe_v, ((c0 + b) % _COLS_SPLIT) * _CC)
        issue_scatter(c0 + b, b)
    for b in range(_NBUF):
        wait_scatter(b)


def kernel(x, pe):
    bsz, seq_len, d_model = x.shape
    x2 = x.reshape(bsz, _ROW)
    pe1 = pe.reshape(-1)
    pe2 = pe.reshape(1, _ROW)

    sc = pl.kernel(
        _sc_add,
        out_type=jax.ShapeDtypeStruct((_B - _N_TC, _ROW), x.dtype),
        mesh=plsc.VectorSubcoreMesh(core_axis_name="c", subcore_axis_name="s"),
        compiler_params=pltpu.CompilerParams(use_tc_tiling_on_sc=True),
        scratch_types=[
            pltpu.VMEM((_ROW,), jnp.float32),
            [pltpu.VMEM((_CR, _CC), jnp.float32) for _ in range(_NBUF)],
            [pltpu.SemaphoreType.DMA for _ in range(_NBUF)],
            [pltpu.SemaphoreType.DMA for _ in range(_NBUF)],
        ],
    )
    out_sc = sc(x2, pe1)

    out_full = pl.pallas_call(
        _tc_add_kernel,
        grid=(_N_TC // _TC_BLOCK,),
        in_specs=[
            pl.BlockSpec((_TC_BLOCK, _ROW), lambda i: (i, 0)),
            pl.BlockSpec((1, _ROW), lambda i: (0, 0)),
        ],
        out_specs=pl.BlockSpec((_TC_BLOCK, _ROW), lambda i: (i, 0)),
        out_shape=jax.ShapeDtypeStruct((_B, _ROW), x.dtype),
    )(x2, pe2)

    out = jax.lax.dynamic_update_slice(out_full, out_sc, (_N_TC, 0))
    return out.reshape(bsz, seq_len, d_model)


# speedup vs baseline: 1.0489x; 1.0163x over previous
import jax
import jax.numpy as jnp
from jax import lax
from jax.experimental import pallas as pl
from jax.experimental.pallas import tpu as pltpu
from jax.experimental.pallas import tpu_sc as plsc

_ROW = 200 * 64
_B = 4096
_N_TC = 3584
_TC_BLOCK = 128

_NC = 2
_NS = 16
_NW = _NC * _NS
_RPW = (_B - _N_TC) // _NW
_CR = 8
_CC = _ROW // 2
_COLS_SPLIT = _ROW // _CC
_CHUNKS = (_RPW // _CR) * _COLS_SPLIT
_NBUF = 2


def _tc_add_kernel(x_ref, pe_ref, o_ref):
    o_ref[...] = x_ref[...] + pe_ref[...]


def _add_pe_chunk(buf, pe_v, col_base):
    @pl.loop(0, _CC // 16, unroll=2)
    def _(j):
        sl = pl.ds(col_base + j * 16, 16)
        pe_row = pe_v[sl]
        for r in range(_CR):
            buf[r, pl.ds(j * 16, 16)] = buf[r, pl.ds(j * 16, 16)] + pe_row


def _sc_add(x_hbm, pe_hbm, out_hbm, pe_v, bufs, gsem, ssem):
    wid = lax.axis_index("s") * _NC + lax.axis_index("c")
    row0 = _N_TC + wid * _RPW

    pltpu.sync_copy(pe_hbm, pe_v)

    def chunk_slice(ref, c, base_row):
        rb = c // _COLS_SPLIT
        h = c % _COLS_SPLIT
        return ref.at[pl.ds(base_row + rb * _CR, _CR), pl.ds(h * _CC, _CC)]

    def issue_gather(c, b):
        pltpu.async_copy(chunk_slice(x_hbm, c, row0), bufs[b], gsem[b])

    def wait_gather(b):
        pltpu.make_async_copy(
            x_hbm.at[pl.ds(0, _CR), pl.ds(0, _CC)], bufs[b], gsem[b]
        ).wait()

    def issue_scatter(c, b):
        pltpu.async_copy(bufs[b], chunk_slice(out_hbm, c, wid * _RPW), ssem[b])

    def wait_scatter(b):
        pltpu.make_async_copy(
            bufs[b], out_hbm.at[pl.ds(0, _CR), pl.ds(0, _CC)], ssem[b]
        ).wait()

    for b in range(_NBUF):
        issue_gather(b, b)

    @pl.loop(0, _CHUNKS // _NBUF - 1)
    def _(s):
        c0 = s * _NBUF
        for b in range(_NBUF):
            wait_gather(b)
            _add_pe_chunk(bufs[b], pe_v, ((c0 + b) % _COLS_SPLIT) * _CC)
            issue_scatter(c0 + b, b)
        for b in range(_NBUF):
            wait_scatter(b)
            issue_gather(c0 + _NBUF + b, b)

    c0 = _CHUNKS - _NBUF
    for b in range(_NBUF):
        wait_gather(b)
        _add_pe_chunk(bufs[b], pe_v, ((c0 + b) % _COLS_SPLIT) * _CC)
        issue_scatter(c0 + b, b)
    for b in range(_NBUF):
        wait_scatter(b)


def kernel(x, pe):
    bsz, seq_len, d_model = x.shape
    x2 = x.reshape(bsz, _ROW)
    pe1 = pe.reshape(-1)
    pe2 = pe.reshape(1, _ROW)

    sc = pl.kernel(
        _sc_add,
        out_type=jax.ShapeDtypeStruct((_B - _N_TC, _ROW), x.dtype),
        mesh=plsc.VectorSubcoreMesh(core_axis_name="c", subcore_axis_name="s"),
        compiler_params=pltpu.CompilerParams(use_tc_tiling_on_sc=True),
        scratch_types=[
            pltpu.VMEM((_ROW,), jnp.float32),
            [pltpu.VMEM((_CR, _CC), jnp.float32) for _ in range(_NBUF)],
            [pltpu.SemaphoreType.DMA for _ in range(_NBUF)],
            [pltpu.SemaphoreType.DMA for _ in range(_NBUF)],
        ],
    )
    out_sc = sc(x2, pe1)

    out_full = pl.pallas_call(
        _tc_add_kernel,
        grid=(_N_TC // _TC_BLOCK,),
        in_specs=[
            pl.BlockSpec((_TC_BLOCK, _ROW), lambda i: (i, 0)),
            pl.BlockSpec((1, _ROW), lambda i: (0, 0)),
        ],
        out_specs=pl.BlockSpec((_TC_BLOCK, _ROW), lambda i: (i, 0)),
        out_shape=jax.ShapeDtypeStruct((_B, _ROW), x.dtype),
    )(x2, pe2)

    out = jax.lax.dynamic_update_slice(out_full, out_sc, (_N_TC, 0))
    return out.reshape(bsz, seq_len, d_model)


# speedup vs baseline: 1.0683x; 1.0185x over previous
import jax
import jax.numpy as jnp
from jax import lax
from jax.experimental import pallas as pl
from jax.experimental.pallas import tpu as pltpu
from jax.experimental.pallas import tpu_sc as plsc

_ROW = 200 * 64
_B = 4096
_N_TC = 3840
_TC_BLOCK = 128

_NC = 2
_NS = 16
_NW = _NC * _NS
_RPW = (_B - _N_TC) // _NW
_CR = 8
_CC = _ROW // 2
_COLS_SPLIT = _ROW // _CC
_CHUNKS = (_RPW // _CR) * _COLS_SPLIT
_NBUF = 2


def _tc_add_kernel(x_ref, pe_ref, o_ref):
    o_ref[...] = x_ref[...] + pe_ref[...]


def _add_pe_chunk(buf, pe_v, col_base):
    @pl.loop(0, _CC // 16, unroll=2)
    def _(j):
        sl = pl.ds(col_base + j * 16, 16)
        pe_row = pe_v[sl]
        for r in range(_CR):
            buf[r, pl.ds(j * 16, 16)] = buf[r, pl.ds(j * 16, 16)] + pe_row


def _sc_add(x_hbm, pe_hbm, out_hbm, pe_v, bufs, gsem, ssem):
    wid = lax.axis_index("s") * _NC + lax.axis_index("c")
    row0 = _N_TC + wid * _RPW

    pltpu.sync_copy(pe_hbm, pe_v)

    def chunk_slice(ref, c, base_row):
        rb = c // _COLS_SPLIT
        h = c % _COLS_SPLIT
        return ref.at[pl.ds(base_row + rb * _CR, _CR), pl.ds(h * _CC, _CC)]

    def issue_gather(c, b):
        pltpu.async_copy(chunk_slice(x_hbm, c, row0), bufs[b], gsem[b])

    def wait_gather(b):
        pltpu.make_async_copy(
            x_hbm.at[pl.ds(0, _CR), pl.ds(0, _CC)], bufs[b], gsem[b]
        ).wait()

    def issue_scatter(c, b):
        pltpu.async_copy(bufs[b], chunk_slice(out_hbm, c, wid * _RPW), ssem[b])

    def wait_scatter(b):
        pltpu.make_async_copy(
            bufs[b], out_hbm.at[pl.ds(0, _CR), pl.ds(0, _CC)], ssem[b]
        ).wait()

    for b in range(_NBUF):
        issue_gather(b, b)

    @pl.loop(0, _CHUNKS // _NBUF - 1)
    def _(s):
        c0 = s * _NBUF
        for b in range(_NBUF):
            wait_gather(b)
            _add_pe_chunk(bufs[b], pe_v, ((c0 + b) % _COLS_SPLIT) * _CC)
            issue_scatter(c0 + b, b)
        for b in range(_NBUF):
            wait_scatter(b)
            issue_gather(c0 + _NBUF + b, b)

    c0 = _CHUNKS - _NBUF
    for b in range(_NBUF):
        wait_gather(b)
        _add_pe_chunk(bufs[b], pe_v, ((c0 + b) % _COLS_SPLIT) * _CC)
        issue_scatter(c0 + b, b)
    for b in range(_NBUF):
        wait_scatter(b)


def kernel(x, pe):
    bsz, seq_len, d_model = x.shape
    x2 = x.reshape(bsz, _ROW)
    pe1 = pe.reshape(-1)
    pe2 = pe.reshape(1, _ROW)

    sc = pl.kernel(
        _sc_add,
        out_type=jax.ShapeDtypeStruct((_B - _N_TC, _ROW), x.dtype),
        mesh=plsc.VectorSubcoreMesh(core_axis_name="c", subcore_axis_name="s"),
        compiler_params=pltpu.CompilerParams(use_tc_tiling_on_sc=True),
        scratch_types=[
            pltpu.VMEM((_ROW,), jnp.float32),
            [pltpu.VMEM((_CR, _CC), jnp.float32) for _ in range(_NBUF)],
            [pltpu.SemaphoreType.DMA for _ in range(_NBUF)],
            [pltpu.SemaphoreType.DMA for _ in range(_NBUF)],
        ],
    )
    out_sc = sc(x2, pe1)

    out_full = pl.pallas_call(
        _tc_add_kernel,
        grid=(_N_TC // _TC_BLOCK,),
        in_specs=[
            pl.BlockSpec((_TC_BLOCK, _ROW), lambda i: (i, 0)),
            pl.BlockSpec((1, _ROW), lambda i: (0, 0)),
        ],
        out_specs=pl.BlockSpec((_TC_BLOCK, _ROW), lambda i: (i, 0)),
        out_shape=jax.ShapeDtypeStruct((_B, _ROW), x.dtype),
    )(x2, pe2)

    out = jax.lax.dynamic_update_slice(out_full, out_sc, (_N_TC, 0))
    return out.reshape(bsz, seq_len, d_model)


# speedup vs baseline: 1.0695x; 1.0011x over previous
import jax
import jax.numpy as jnp
from jax import lax
from jax.experimental import pallas as pl
from jax.experimental.pallas import tpu as pltpu
from jax.experimental.pallas import tpu_sc as plsc

_ROW = 200 * 64
_B = 4096
_N_TC = 3840
_TC_BLOCK = 256

_NC = 2
_NS = 16
_NW = _NC * _NS
_RPW = (_B - _N_TC) // _NW
_CR = 8
_CC = _ROW // 2
_COLS_SPLIT = _ROW // _CC
_CHUNKS = (_RPW // _CR) * _COLS_SPLIT
_NBUF = 2


def _tc_add_kernel(x_ref, pe_ref, o_ref):
    o_ref[...] = x_ref[...] + pe_ref[...]


def _add_pe_chunk(buf, pe_v, col_base):
    @pl.loop(0, _CC // 16, unroll=2)
    def _(j):
        sl = pl.ds(col_base + j * 16, 16)
        pe_row = pe_v[sl]
        for r in range(_CR):
            buf[r, pl.ds(j * 16, 16)] = buf[r, pl.ds(j * 16, 16)] + pe_row


def _sc_add(x_hbm, pe_hbm, out_hbm, pe_v, bufs, gsem, ssem):
    wid = lax.axis_index("s") * _NC + lax.axis_index("c")
    row0 = _N_TC + wid * _RPW

    pltpu.sync_copy(pe_hbm, pe_v)

    def chunk_slice(ref, c, base_row):
        rb = c // _COLS_SPLIT
        h = c % _COLS_SPLIT
        return ref.at[pl.ds(base_row + rb * _CR, _CR), pl.ds(h * _CC, _CC)]

    def issue_gather(c, b):
        pltpu.async_copy(chunk_slice(x_hbm, c, row0), bufs[b], gsem[b])

    def wait_gather(b):
        pltpu.make_async_copy(
            x_hbm.at[pl.ds(0, _CR), pl.ds(0, _CC)], bufs[b], gsem[b]
        ).wait()

    def issue_scatter(c, b):
        pltpu.async_copy(bufs[b], chunk_slice(out_hbm, c, wid * _RPW), ssem[b])

    def wait_scatter(b):
        pltpu.make_async_copy(
            bufs[b], out_hbm.at[pl.ds(0, _CR), pl.ds(0, _CC)], ssem[b]
        ).wait()

    for b in range(_NBUF):
        issue_gather(b, b)

    @pl.loop(0, _CHUNKS // _NBUF - 1)
    def _(s):
        c0 = s * _NBUF
        for b in range(_NBUF):
            wait_gather(b)
            _add_pe_chunk(bufs[b], pe_v, ((c0 + b) % _COLS_SPLIT) * _CC)
            issue_scatter(c0 + b, b)
        for b in range(_NBUF):
            wait_scatter(b)
            issue_gather(c0 + _NBUF + b, b)

    c0 = _CHUNKS - _NBUF
    for b in range(_NBUF):
        wait_gather(b)
        _add_pe_chunk(bufs[b], pe_v, ((c0 + b) % _COLS_SPLIT) * _CC)
        issue_scatter(c0 + b, b)
    for b in range(_NBUF):
        wait_scatter(b)


def kernel(x, pe):
    bsz, seq_len, d_model = x.shape
    x2 = x.reshape(bsz, _ROW)
    pe1 = pe.reshape(-1)
    pe2 = pe.reshape(1, _ROW)

    sc = pl.kernel(
        _sc_add,
        out_type=jax.ShapeDtypeStruct((_B - _N_TC, _ROW), x.dtype),
        mesh=plsc.VectorSubcoreMesh(core_axis_name="c", subcore_axis_name="s"),
        compiler_params=pltpu.CompilerParams(use_tc_tiling_on_sc=True),
        scratch_types=[
            pltpu.VMEM((_ROW,), jnp.float32),
            [pltpu.VMEM((_CR, _CC), jnp.float32) for _ in range(_NBUF)],
            [pltpu.SemaphoreType.DMA for _ in range(_NBUF)],
            [pltpu.SemaphoreType.DMA for _ in range(_NBUF)],
        ],
    )
    out_sc = sc(x2, pe1)

    out_full = pl.pallas_call(
        _tc_add_kernel,
        grid=(_N_TC // _TC_BLOCK,),
        in_specs=[
            pl.BlockSpec((_TC_BLOCK, _ROW), lambda i: (i, 0)),
            pl.BlockSpec((1, _ROW), lambda i: (0, 0)),
        ],
        out_specs=pl.BlockSpec((_TC_BLOCK, _ROW), lambda i: (i, 0)),
        out_shape=jax.ShapeDtypeStruct((_B, _ROW), x.dtype),
    )(x2, pe2)

    out = jax.lax.dynamic_update_slice(out_full, out_sc, (_N_TC, 0))
    return out.reshape(bsz, seq_len, d_model)
